# Initial kernel scaffold; baseline (speedup 1.0000x reference)
import jax
import jax.numpy as jnp
from jax.experimental import pallas as pl


def kernel(features, edge_index, W1, b1, W2, b2, W3, b3):
    raise NotImplementedError("write your pallas kernel here")


# speedup vs baseline: 4.3236x; 4.3236x over previous
import functools

import jax
import jax.numpy as jnp
from jax import lax
from jax.experimental import pallas as pl
from jax.experimental.pallas import tpu as pltpu
from jax.experimental.pallas import tpu_sc as plsc

_N = 10000
_E = 320000
_NC = 2
_NS = 16
_NW = _NC * _NS
_EPW = _E // _NW
_CHUNK = 80
_NPAD = 10112
_RPS = _NPAD // _NS

_BN = 1000


def _mesh():
    return plsc.VectorSubcoreMesh(core_axis_name="c", subcore_axis_name="s")


def _sc_degrees(src, dst, ones_rows, k_rows, zeros_rows):
    out_type = jax.ShapeDtypeStruct((_NC, _NPAD, 128), jnp.float32)

    @functools.partial(
        pl.kernel, mesh=_mesh(), out_type=out_type,
        scratch_types=[
            pltpu.VMEM((_CHUNK,), jnp.int32),
            pltpu.VMEM((_CHUNK,), jnp.int32),
            pltpu.VMEM((_CHUNK, 128), jnp.float32),
            pltpu.VMEM((_CHUNK, 128), jnp.float32),
            pltpu.VMEM_SHARED((_NPAD, 128), jnp.float32),
        ])
    def k(src_hbm, dst_hbm, ones_hbm, k_hbm, zeros_hbm, out_hbm,
          sidx, didx, ones_v, k_v, acc):
        cid = lax.axis_index("c")
        sid = lax.axis_index("s")
        wid = sid * _NC + cid
        r0 = sid * _RPS
        pltpu.sync_copy(ones_hbm, ones_v)
        pltpu.sync_copy(k_hbm, k_v)
        pltpu.sync_copy(zeros_hbm, acc.at[pl.ds(r0, _RPS)])
        plsc.subcore_barrier()
        base0 = wid * _EPW

        @pl.loop(0, _EPW, step=_CHUNK)
        def _(j):
            pltpu.sync_copy(src_hbm.at[pl.ds(base0 + j, _CHUNK)], sidx)
            pltpu.sync_copy(dst_hbm.at[pl.ds(base0 + j, _CHUNK)], didx)
            pltpu.sync_copy(ones_v, acc.at[sidx], add=True)
            pltpu.sync_copy(k_v, acc.at[didx], add=True)

        plsc.subcore_barrier()
        pltpu.sync_copy(acc.at[pl.ds(r0, _RPS)],
                        out_hbm.at[cid, pl.ds(r0, _RPS)])

    return k(src, dst, ones_rows, k_rows, zeros_rows)


def _sc_aggregate(z, src, dst, zeros_rows):
    f = z.shape[1]
    out_type = jax.ShapeDtypeStruct((_NC, _NPAD, f), jnp.float32)

    @functools.partial(
        pl.kernel, mesh=_mesh(), out_type=out_type,
        scratch_types=[
            pltpu.VMEM((_CHUNK,), jnp.int32),
            pltpu.VMEM((_CHUNK,), jnp.int32),
            pltpu.VMEM((_CHUNK, f), jnp.float32),
            pltpu.VMEM_SHARED((_NPAD, f), jnp.float32),
            pltpu.SemaphoreType.DMA,
        ])
    def k(z_hbm, src_hbm, dst_hbm, zeros_hbm, out_hbm,
          sidx, didx, rows_v, acc, sem):
        cid = lax.axis_index("c")
        sid = lax.axis_index("s")
        wid = sid * _NC + cid
        r0 = sid * _RPS
        pltpu.sync_copy(zeros_hbm, acc.at[pl.ds(r0, _RPS)])
        plsc.subcore_barrier()
        base0 = wid * _EPW

        @pl.loop(0, _EPW, step=_CHUNK)
        def _(j):
            pltpu.sync_copy(src_hbm.at[pl.ds(base0 + j, _CHUNK)], sidx)
            pltpu.sync_copy(dst_hbm.at[pl.ds(base0 + j, _CHUNK)], didx)
            pltpu.async_copy(z_hbm.at[sidx], rows_v, sem).wait()
            pltpu.sync_copy(rows_v, acc.at[didx], add=True)

        plsc.subcore_barrier()
        pltpu.sync_copy(acc.at[pl.ds(r0, _RPS)],
                        out_hbm.at[cid, pl.ds(r0, _RPS)])

    return k(z, src, dst, zeros_rows)


def _tc_prep(features, h0, h1, w1):
    grid = (_N // _BN,)

    def body(f_ref, a_ref, b_ref, w_ref, ns_ref, nd_ref, z_ref):
        v = a_ref[:, 0:1] + b_ref[:, 0:1]
        deg_in = jnp.floor(v * (1.0 / 65536.0))
        deg_out = v - deg_in * 65536.0
        ns = lax.rsqrt(jnp.maximum(deg_out, 1.0))
        nd = lax.rsqrt(jnp.maximum(deg_in, 1.0))
        ns_ref[...] = ns
        nd_ref[...] = nd
        z_ref[...] = jnp.dot(f_ref[...] * ns, w_ref[...],
                             preferred_element_type=jnp.float32)

    row = lambda i: (i, 0)
    fixed = lambda i: (0, 0)
    return pl.pallas_call(
        body,
        grid=grid,
        in_specs=[
            pl.BlockSpec((_BN, 128), row),
            pl.BlockSpec((_BN, 128), row),
            pl.BlockSpec((_BN, 128), row),
            pl.BlockSpec((128, 128), fixed),
        ],
        out_specs=[
            pl.BlockSpec((_BN, 1), row),
            pl.BlockSpec((_BN, 1), row),
            pl.BlockSpec((_BN, 128), row),
        ],
        out_shape=[
            jax.ShapeDtypeStruct((_N, 1), jnp.float32),
            jax.ShapeDtypeStruct((_N, 1), jnp.float32),
            jax.ShapeDtypeStruct((_N, 128), jnp.float32),
        ],
    )(features, h0, h1, w1)


def _tc_layer(p0, p1, bias, nd, ns, w_next):
    f = p0.shape[1]
    f_next = w_next.shape[1]
    grid = (_N // _BN,)

    def body(p0_ref, p1_ref, b_ref, nd_ref, ns_ref, w_ref, z_ref):
        agg = p0_ref[...] + p1_ref[...]
        h = jnp.maximum(nd_ref[...] * agg + b_ref[...], 0.0)
        z_ref[...] = jnp.dot(ns_ref[...] * h, w_ref[...],
                             preferred_element_type=jnp.float32)

    row = lambda i: (i, 0)
    fixed = lambda i: (0, 0)
    return pl.pallas_call(
        body,
        grid=grid,
        in_specs=[
            pl.BlockSpec((_BN, f), row),
            pl.BlockSpec((_BN, f), row),
            pl.BlockSpec((1, f), fixed),
            pl.BlockSpec((_BN, 1), row),
            pl.BlockSpec((_BN, 1), row),
            pl.BlockSpec((f, f_next), fixed),
        ],
        out_specs=pl.BlockSpec((_BN, f_next), row),
        out_shape=jax.ShapeDtypeStruct((_N, f_next), jnp.float32),
    )(p0, p1, bias, nd, ns, w_next)


def _tc_layer_noproj(p0, p1, bias, nd, ns):
    f = p0.shape[1]
    grid = (_N // _BN,)

    def body(p0_ref, p1_ref, b_ref, nd_ref, ns_ref, y_ref):
        agg = p0_ref[...] + p1_ref[...]
        h = jnp.maximum(nd_ref[...] * agg + b_ref[...], 0.0)
        y_ref[...] = ns_ref[...] * h

    row = lambda i: (i, 0)
    fixed = lambda i: (0, 0)
    return pl.pallas_call(
        body,
        grid=grid,
        in_specs=[
            pl.BlockSpec((_BN, f), row),
            pl.BlockSpec((_BN, f), row),
            pl.BlockSpec((1, f), fixed),
            pl.BlockSpec((_BN, 1), row),
            pl.BlockSpec((_BN, 1), row),
        ],
        out_specs=pl.BlockSpec((_BN, f), row),
        out_shape=jax.ShapeDtypeStruct((_N, f), jnp.float32),
    )(p0, p1, bias, nd, ns)


def _tc_final(p0, p1, bias, nd, w3):
    f = p0.shape[1]
    f_out = w3.shape[1]
    grid = (_N // _BN,)

    def body(p0_ref, p1_ref, b_ref, nd_ref, w_ref, o_ref):
        agg = p0_ref[...] + p1_ref[...]
        t = jnp.dot(nd_ref[...] * agg, w_ref[...],
                    preferred_element_type=jnp.float32) + b_ref[...]
        o_ref[...] = jax.nn.sigmoid(t) + 1e-8

    row = lambda i: (i, 0)
    fixed = lambda i: (0, 0)
    return pl.pallas_call(
        body,
        grid=grid,
        in_specs=[
            pl.BlockSpec((_BN, f), row),
            pl.BlockSpec((_BN, f), row),
            pl.BlockSpec((1, f_out), fixed),
            pl.BlockSpec((_BN, 1), row),
            pl.BlockSpec((f, f_out), fixed),
        ],
        out_specs=pl.BlockSpec((_BN, f_out), row),
        out_shape=jax.ShapeDtypeStruct((_N, f_out), jnp.float32),
    )(p0, p1, bias, nd, w3)


def kernel(features, edge_index, W1, b1, W2, b2, W3, b3):
    src = edge_index[0]
    dst = edge_index[1]
    ones_rows = jnp.ones((_CHUNK, 128), jnp.float32)
    k_rows = jnp.full((_CHUNK, 128), 65536.0, jnp.float32)
    zeros128 = jnp.zeros((_RPS, 128), jnp.float32)

    hist = _sc_degrees(src, dst, ones_rows, k_rows, zeros128)
    ns, nd, z1 = _tc_prep(features, hist[0, :_N], hist[1, :_N], W1)
    p = _sc_aggregate(z1, src, dst, zeros128)
    z2 = _tc_layer(p[0, :_N], p[1, :_N], b1.reshape(1, -1), nd, ns, W2)
    p = _sc_aggregate(z2, src, dst, zeros128)
    y3 = _tc_layer_noproj(p[0, :_N], p[1, :_N], b2.reshape(1, -1), nd, ns)
    p = _sc_aggregate(y3, src, dst, zeros128)
    return _tc_final(p[0, :_N], p[1, :_N], b3.reshape(1, -1), nd, W3)
